# baseline (speedup 1.0000x reference)
import functools

import jax
import jax.numpy as jnp
from jax import lax
from jax.experimental import pallas as pl
from jax.experimental.pallas import tpu as pltpu
from jax.experimental.pallas import tpu_sc as plsc

N_CORES = 2
N_SUBCORES = 16
N_WORKERS = N_CORES * N_SUBCORES
LANES = 16


def _wid():
    return lax.axis_index("s") * N_CORES + lax.axis_index("c")


def _sc_gather(inputs, idx, gates_flat, k):
    n_tok, d = inputs.shape
    nk = idx.shape[0]
    ch = nk // N_WORKERS
    mesh = plsc.VectorSubcoreMesh(core_axis_name="c", subcore_axis_name="s")

    @functools.partial(
        pl.kernel,
        mesh=mesh,
        out_type=(
            jax.ShapeDtypeStruct((nk, d), inputs.dtype),
            jax.ShapeDtypeStruct((nk,), jnp.float32),
        ),
        scratch_types=[
            pltpu.VMEM((ch,), jnp.int32),
            pltpu.VMEM((ch,), jnp.int32),
            pltpu.VMEM((nk,), jnp.float32),
            pltpu.VMEM((ch,), jnp.float32),
            pltpu.VMEM((ch, d), inputs.dtype),
        ] + [pltpu.SemaphoreType.DMA] * 8,
        compiler_params=pltpu.CompilerParams(needs_layout_passes=False),
    )
    def gather_k(x_hbm, idx_hbm, g_hbm, xs_hbm, gs_hbm,
                 idx_v, tok_v, g_v, gout_v, rows_v, *sems):
        base = _wid() * ch
        n_chunks = 4
        hc = ch // n_chunks
        pltpu.sync_copy(idx_hbm.at[pl.ds(base, ch)], idx_v)
        for j in range(ch // LANES):
            sl = pl.ds(j * LANES, LANES)
            tok_v[sl] = idx_v[sl] // k
        cps = []
        for h in range(n_chunks):
            rows = pl.ds(h * hc, hc)
            cps.append(pltpu.async_copy(
                x_hbm.at[tok_v.at[rows]], rows_v.at[rows], sems[h]))
        pltpu.sync_copy(g_hbm, g_v)
        for j in range(ch // LANES):
            sl = pl.ds(j * LANES, LANES)
            gout_v[sl] = plsc.load_gather(g_v, [idx_v[sl]])
        wbs = []
        for h in range(n_chunks):
            cps[h].wait()
            wbs.append(pltpu.async_copy(
                rows_v.at[pl.ds(h * hc, hc)],
                xs_hbm.at[pl.ds(base + h * hc, hc)], sems[n_chunks + h]))
        pltpu.sync_copy(gout_v, gs_hbm.at[pl.ds(base, ch)])
        for wb in wbs:
            wb.wait()

    return gather_k(inputs, idx, gates_flat)


def _sc_combine(y, idx, k):
    nk, d = y.shape
    n_tok = nk // k
    ch = n_tok // N_WORKERS
    sch = ch * k
    n_chunks = 4
    hc = ch // n_chunks
    mesh = plsc.VectorSubcoreMesh(core_axis_name="c", subcore_axis_name="s")

    @functools.partial(
        pl.kernel,
        mesh=mesh,
        out_type=jax.ShapeDtypeStruct((n_tok, d), jnp.float32),
        scratch_types=[
            pltpu.VMEM((nk,), jnp.int32),
            pltpu.VMEM((nk,), jnp.int32),
            pltpu.VMEM((ch,), jnp.int32),
            pltpu.VMEM((ch,), jnp.int32),
            pltpu.VMEM((ch, d), jnp.float32),
            pltpu.VMEM((ch, d), jnp.float32),
        ] + [pltpu.SemaphoreType.DMA] * (3 * n_chunks),
        compiler_params=pltpu.CompilerParams(needs_layout_passes=False),
    )
    def combine_k(y_hbm, idx_hbm, out_hbm,
                  idx_v, inv_v, ia_v, ib_v, a_v, b_v, *sems):
        wid = _wid()
        tbase = wid * ch
        sbase = wid * sch
        pltpu.sync_copy(idx_hbm, idx_v)
        lane = lax.broadcasted_iota(jnp.int32, (LANES,), 0)
        for g in range(nk // LANES):
            sl = pl.ds(g * LANES, LANES)
            plsc.store_scatter(inv_v, [idx_v[sl]], g * LANES + lane)
        for g in range(ch // LANES):
            t16 = sbase + k * (g * LANES + lane)
            ia_v[pl.ds(g * LANES, LANES)] = plsc.load_gather(inv_v, [t16])
            ib_v[pl.ds(g * LANES, LANES)] = plsc.load_gather(inv_v, [t16 + 1])

        cps = []
        for h in range(n_chunks):
            rows = pl.ds(h * hc, hc)
            cps.append((
                pltpu.async_copy(y_hbm.at[ia_v.at[rows]], a_v.at[rows], sems[2 * h]),
                pltpu.async_copy(y_hbm.at[ib_v.at[rows]], b_v.at[rows], sems[2 * h + 1]),
            ))

        def add_row(j, carry):
            for c in range(d // LANES):
                sl = pl.ds(c * LANES, LANES)
                a_v[j, sl] = a_v[j, sl] + b_v[j, sl]
            return carry

        wbs = []
        for h in range(n_chunks):
            cps[h][0].wait()
            cps[h][1].wait()
            lax.fori_loop(h * hc, (h + 1) * hc, add_row, 0)
            wbs.append(pltpu.async_copy(
                a_v.at[pl.ds(h * hc, hc)],
                out_hbm.at[pl.ds(tbase + h * hc, hc)], sems[2 * n_chunks + h]))
        for wb in wbs:
            wb.wait()

    return combine_k(y, idx)


def _tc_gmm(xp, gate, weight, offsets, block_rows=512):
    nk, d_half = xp.shape
    n_exp, d_out, d_in = weight.shape
    bt = block_rows
    n_tiles = nk // bt

    def body(x_ref, g_ref, w_ref, off_ref, o_ref):
        t = pl.program_id(0)
        base = t * bt
        u = lax.bitcast_convert_type(x_ref[...], jnp.uint32)
        lo = lax.bitcast_convert_type(
            (u & jnp.uint32(0xFFFF)).astype(jnp.uint16), jnp.bfloat16)
        hi = lax.bitcast_convert_type(
            (u >> jnp.uint32(16)).astype(jnp.uint16), jnp.bfloat16)
        xu = jnp.concatenate([lo, hi], axis=1).astype(jnp.float32)
        xs = xu * g_ref[...].reshape(bt, 1)
        gi = base + lax.broadcasted_iota(jnp.int32, (bt, 1), 0)
        e_first = jnp.int32(0)
        e_last1 = jnp.int32(1)
        for e in range(n_exp):
            e_first = e_first + jnp.where(off_ref[e] <= base, 1, 0).astype(jnp.int32)
            if e < n_exp - 1:
                e_last1 = e_last1 + jnp.where(off_ref[e] < base + bt, 1, 0).astype(jnp.int32)

        o_ref[...] = jnp.zeros((bt, d_out), jnp.float32)

        def do_expert(e, carry):
            start = jnp.where(e == 0, 0, off_ref[jnp.maximum(e - 1, 0)])
            end = off_ref[e]
            m = (gi >= start) & (gi < end)
            xm = jnp.where(m, xs, 0.0)
            o_ref[...] += lax.dot_general(
                xm, w_ref[e],
                (((1,), (1,)), ((), ())),
                preferred_element_type=jnp.float32,
            )
            return carry

        lax.fori_loop(e_first, e_last1, do_expert, 0)

    return pl.pallas_call(
        body,
        grid=(n_tiles,),
        in_specs=[
            pl.BlockSpec((bt, d_half), lambda t: (t, 0)),
            pl.BlockSpec((bt,), lambda t: (t,)),
            pl.BlockSpec((n_exp, d_out, d_in), lambda t: (0, 0, 0)),
            pl.BlockSpec(memory_space=pltpu.SMEM),
        ],
        out_specs=pl.BlockSpec((bt, d_out), lambda t: (t, 0)),
        out_shape=jax.ShapeDtypeStruct((nk, d_out), jnp.float32),
        compiler_params=pltpu.CompilerParams(
            dimension_semantics=("arbitrary",),
        ),
    )(xp, gate, weight, offsets)


def kernel(inputs, weight, k, sorted_expert_idxs, sorted_scattered_idxs,
           padded_block_idxs, expert_offsets, gates):
    del k, sorted_expert_idxs, padded_block_idxs
    k_static = gates.shape[1]
    idx = sorted_scattered_idxs
    gates_flat = gates.reshape(-1)

    d_in = inputs.shape[1]
    xb = inputs.astype(jnp.bfloat16)
    lo = lax.bitcast_convert_type(xb[:, :d_in // 2], jnp.uint16).astype(jnp.uint32)
    hi = lax.bitcast_convert_type(xb[:, d_in // 2:], jnp.uint16).astype(jnp.uint32)
    xp = lax.bitcast_convert_type(lo | (hi << jnp.uint32(16)), jnp.int32)

    xp_sorted, gate_sorted = _sc_gather(xp, idx, gates_flat, k_static)
    y = _tc_gmm(xp_sorted, gate_sorted, weight, expert_offsets)
    return _sc_combine(y, idx, k_static)


# speedup vs baseline: 1.0054x; 1.0054x over previous
import functools

import jax
import jax.numpy as jnp
from jax import lax
from jax.experimental import pallas as pl
from jax.experimental.pallas import tpu as pltpu
from jax.experimental.pallas import tpu_sc as plsc

N_CORES = 2
N_SUBCORES = 16
N_WORKERS = N_CORES * N_SUBCORES
LANES = 16


def _wid():
    return lax.axis_index("s") * N_CORES + lax.axis_index("c")


def _sc_gather(inputs, idx, gates_flat, k):
    n_tok, d = inputs.shape
    nk = idx.shape[0]
    ch = nk // N_WORKERS
    mesh = plsc.VectorSubcoreMesh(core_axis_name="c", subcore_axis_name="s")

    @functools.partial(
        pl.kernel,
        mesh=mesh,
        out_type=(
            jax.ShapeDtypeStruct((nk, d), inputs.dtype),
            jax.ShapeDtypeStruct((nk,), jnp.float32),
        ),
        scratch_types=[
            pltpu.VMEM((ch,), jnp.int32),
            pltpu.VMEM((ch,), jnp.int32),
            pltpu.VMEM((nk,), jnp.float32),
            pltpu.VMEM((ch,), jnp.float32),
            pltpu.VMEM((ch, d), inputs.dtype),
        ] + [pltpu.SemaphoreType.DMA] * 8,
        compiler_params=pltpu.CompilerParams(needs_layout_passes=False),
    )
    def gather_k(x_hbm, idx_hbm, g_hbm, xs_hbm, gs_hbm,
                 idx_v, tok_v, g_v, gout_v, rows_v, *sems):
        base = _wid() * ch
        n_chunks = 2
        hc = ch // n_chunks
        pltpu.sync_copy(idx_hbm.at[pl.ds(base, ch)], idx_v)
        for j in range(ch // LANES):
            sl = pl.ds(j * LANES, LANES)
            tok_v[sl] = idx_v[sl] // k
        cps = []
        for h in range(n_chunks):
            rows = pl.ds(h * hc, hc)
            cps.append(pltpu.async_copy(
                x_hbm.at[tok_v.at[rows]], rows_v.at[rows], sems[h]))
        pltpu.sync_copy(g_hbm, g_v)
        for j in range(ch // LANES):
            sl = pl.ds(j * LANES, LANES)
            gout_v[sl] = plsc.load_gather(g_v, [idx_v[sl]])
        wbs = []
        for h in range(n_chunks):
            cps[h].wait()
            wbs.append(pltpu.async_copy(
                rows_v.at[pl.ds(h * hc, hc)],
                xs_hbm.at[pl.ds(base + h * hc, hc)], sems[n_chunks + h]))
        pltpu.sync_copy(gout_v, gs_hbm.at[pl.ds(base, ch)])
        for wb in wbs:
            wb.wait()

    return gather_k(inputs, idx, gates_flat)


def _sc_combine(y, idx, k):
    nk, d = y.shape
    n_tok = nk // k
    ch = n_tok // N_WORKERS
    sch = ch * k
    n_chunks = 4
    hc = ch // n_chunks
    mesh = plsc.VectorSubcoreMesh(core_axis_name="c", subcore_axis_name="s")

    @functools.partial(
        pl.kernel,
        mesh=mesh,
        out_type=jax.ShapeDtypeStruct((n_tok, d), jnp.float32),
        scratch_types=[
            pltpu.VMEM((nk,), jnp.int32),
            pltpu.VMEM((nk,), jnp.int32),
            pltpu.VMEM((ch,), jnp.int32),
            pltpu.VMEM((ch,), jnp.int32),
            pltpu.VMEM((ch, d), jnp.float32),
            pltpu.VMEM((ch, d), jnp.float32),
        ] + [pltpu.SemaphoreType.DMA] * (3 * n_chunks),
        compiler_params=pltpu.CompilerParams(needs_layout_passes=False),
    )
    def combine_k(y_hbm, idx_hbm, out_hbm,
                  idx_v, inv_v, ia_v, ib_v, a_v, b_v, *sems):
        wid = _wid()
        tbase = wid * ch
        sbase = wid * sch
        pltpu.sync_copy(idx_hbm, idx_v)
        lane = lax.broadcasted_iota(jnp.int32, (LANES,), 0)
        for g in range(nk // LANES):
            sl = pl.ds(g * LANES, LANES)
            plsc.store_scatter(inv_v, [idx_v[sl]], g * LANES + lane)
        for g in range(ch // LANES):
            t16 = sbase + k * (g * LANES + lane)
            ia_v[pl.ds(g * LANES, LANES)] = plsc.load_gather(inv_v, [t16])
            ib_v[pl.ds(g * LANES, LANES)] = plsc.load_gather(inv_v, [t16 + 1])

        cps = []
        for h in range(n_chunks):
            rows = pl.ds(h * hc, hc)
            cps.append((
                pltpu.async_copy(y_hbm.at[ia_v.at[rows]], a_v.at[rows], sems[2 * h]),
                pltpu.async_copy(y_hbm.at[ib_v.at[rows]], b_v.at[rows], sems[2 * h + 1]),
            ))

        def add_row(j, carry):
            for c in range(d // LANES):
                sl = pl.ds(c * LANES, LANES)
                a_v[j, sl] = a_v[j, sl] + b_v[j, sl]
            return carry

        wbs = []
        for h in range(n_chunks):
            cps[h][0].wait()
            cps[h][1].wait()
            lax.fori_loop(h * hc, (h + 1) * hc, add_row, 0)
            wbs.append(pltpu.async_copy(
                a_v.at[pl.ds(h * hc, hc)],
                out_hbm.at[pl.ds(tbase + h * hc, hc)], sems[2 * n_chunks + h]))
        for wb in wbs:
            wb.wait()

    return combine_k(y, idx)


def _tc_gmm(xp, gate, weight, offsets, block_rows=512):
    nk, d_half = xp.shape
    n_exp, d_out, d_in = weight.shape
    bt = block_rows
    n_tiles = nk // bt

    def body(x_ref, g_ref, w_ref, off_ref, o_ref):
        t = pl.program_id(0)
        base = t * bt
        u = lax.bitcast_convert_type(x_ref[...], jnp.uint32)
        lo = lax.bitcast_convert_type(
            (u & jnp.uint32(0xFFFF)).astype(jnp.uint16), jnp.bfloat16)
        hi = lax.bitcast_convert_type(
            (u >> jnp.uint32(16)).astype(jnp.uint16), jnp.bfloat16)
        xu = jnp.concatenate([lo, hi], axis=1).astype(jnp.float32)
        xs = xu * g_ref[...].reshape(bt, 1)
        gi = base + lax.broadcasted_iota(jnp.int32, (bt, 1), 0)
        e_first = jnp.int32(0)
        e_last1 = jnp.int32(1)
        for e in range(n_exp):
            e_first = e_first + jnp.where(off_ref[e] <= base, 1, 0).astype(jnp.int32)
            if e < n_exp - 1:
                e_last1 = e_last1 + jnp.where(off_ref[e] < base + bt, 1, 0).astype(jnp.int32)

        o_ref[...] = jnp.zeros((bt, d_out), jnp.float32)

        def do_expert(e, carry):
            start = jnp.where(e == 0, 0, off_ref[jnp.maximum(e - 1, 0)])
            end = off_ref[e]
            m = (gi >= start) & (gi < end)
            xm = jnp.where(m, xs, 0.0)
            o_ref[...] += lax.dot_general(
                xm, w_ref[e],
                (((1,), (1,)), ((), ())),
                preferred_element_type=jnp.float32,
            )
            return carry

        lax.fori_loop(e_first, e_last1, do_expert, 0)

    return pl.pallas_call(
        body,
        grid=(n_tiles,),
        in_specs=[
            pl.BlockSpec((bt, d_half), lambda t: (t, 0)),
            pl.BlockSpec((bt,), lambda t: (t,)),
            pl.BlockSpec((n_exp, d_out, d_in), lambda t: (0, 0, 0)),
            pl.BlockSpec(memory_space=pltpu.SMEM),
        ],
        out_specs=pl.BlockSpec((bt, d_out), lambda t: (t, 0)),
        out_shape=jax.ShapeDtypeStruct((nk, d_out), jnp.float32),
        compiler_params=pltpu.CompilerParams(
            dimension_semantics=("arbitrary",),
        ),
    )(xp, gate, weight, offsets)


def kernel(inputs, weight, k, sorted_expert_idxs, sorted_scattered_idxs,
           padded_block_idxs, expert_offsets, gates):
    del k, sorted_expert_idxs, padded_block_idxs
    k_static = gates.shape[1]
    idx = sorted_scattered_idxs
    gates_flat = gates.reshape(-1)

    d_in = inputs.shape[1]
    xb = inputs.astype(jnp.bfloat16)
    lo = lax.bitcast_convert_type(xb[:, :d_in // 2], jnp.uint16).astype(jnp.uint32)
    hi = lax.bitcast_convert_type(xb[:, d_in // 2:], jnp.uint16).astype(jnp.uint32)
    xp = lax.bitcast_convert_type(lo | (hi << jnp.uint32(16)), jnp.int32)

    xp_sorted, gate_sorted = _sc_gather(xp, idx, gates_flat, k_static)
    y = _tc_gmm(xp_sorted, gate_sorted, weight, expert_offsets)
    return _sc_combine(y, idx, k_static)
